# baseline (speedup 1.0000x reference)
import functools
import math

import jax
import jax.numpy as jnp
from jax import lax
from jax.experimental import pallas as pl
from jax.experimental.pallas import tpu as pltpu
from jax.experimental.pallas import tpu_sc as plsc

_VOCAB = 50257
_D = 1024
_MAXSEQ = 2048
_LANES = 16
_NC = 2
_NS = 16
_NW = _NC * _NS
_SCALE = math.sqrt(_D)


def _emb_body(x_hbm, wtab_hbm, ptab_hbm, out_hbm, idx_v, w_v, p_v, sem,
              *, rows_per_w, chunk):
    wid = lax.axis_index("s") * _NC + lax.axis_index("c")
    base = wid * rows_per_w
    pos_base = lax.rem(base, _MAXSEQ)
    n_chunks = rows_per_w // chunk
    jcols = _D // _LANES

    def do_chunk(c, _):
        row0 = base + c * chunk
        pos0 = pos_base + c * chunk
        pltpu.sync_copy(x_hbm.at[pl.ds(row0, chunk)], idx_v)
        gather = pltpu.async_copy(wtab_hbm.at[idx_v], w_v, sem)
        pltpu.sync_copy(ptab_hbm.at[pl.ds(pos0, chunk)], p_v)
        gather.wait()

        def do_row(i, _):
            for j in range(jcols):
                sl = pl.ds(j * _LANES, _LANES)
                plsc.addupdate(p_v.at[i, sl], w_v[i, sl] * _SCALE)
            return 0

        lax.fori_loop(0, chunk, do_row, 0)
        pltpu.sync_copy(p_v, out_hbm.at[pl.ds(row0, chunk)])
        return 0

    lax.fori_loop(0, n_chunks, do_chunk, 0)


def kernel(x, word_emb, pos_emb):
    batch, seq = x.shape
    nrows = batch * seq
    rows_per_w = nrows // _NW
    chunk = 32

    mesh = plsc.VectorSubcoreMesh(core_axis_name="c", subcore_axis_name="s")
    body = functools.partial(_emb_body, rows_per_w=rows_per_w, chunk=chunk)
    out = pl.kernel(
        body,
        out_type=jax.ShapeDtypeStruct((nrows, _D), jnp.float32),
        mesh=mesh,
        scratch_types=[
            pltpu.VMEM((chunk,), jnp.int32),
            pltpu.VMEM((chunk, _D), jnp.float32),
            pltpu.VMEM((chunk, _D), jnp.float32),
            pltpu.SemaphoreType.DMA,
        ],
    )(x.reshape(nrows).astype(jnp.int32), word_emb, pos_emb)
    return out.reshape(batch, seq, _D)


# speedup vs baseline: 1.3148x; 1.3148x over previous
import functools
import math

import jax
import jax.numpy as jnp
from jax import lax
from jax.experimental import pallas as pl
from jax.experimental.pallas import tpu as pltpu
from jax.experimental.pallas import tpu_sc as plsc

_D = 1024
_MAXSEQ = 2048
_LANES = 16
_NC = 2
_NS = 16
_NW = _NC * _NS
_SCALE = math.sqrt(_D)
_CHUNK = 16


def _emb_body(x_hbm, wtab_hbm, ptab_hbm, out_hbm,
              idx0, idx1, w0, w1, p0, p1, p2,
              g0, g1, s0, s1, s2, o0, o1, o2,
              *, rows_per_w):
    wid = lax.axis_index("s") * _NC + lax.axis_index("c")
    base = wid * rows_per_w
    pos_base = lax.rem(base, _MAXSEQ)
    n_chunks = rows_per_w // _CHUNK
    jcols = _D // _LANES

    idx = [idx0, idx1]
    w = [w0, w1]
    p = [p0, p1, p2]
    gsem = [g0, g1]
    psem = [s0, s1, s2]
    osem = [o0, o1, o2]

    def start_loads(cc):
        b2, b3 = cc % 2, cc % 3
        pltpu.sync_copy(x_hbm.at[pl.ds(base + cc * _CHUNK, _CHUNK)], idx[b2])
        gd = pltpu.async_copy(wtab_hbm.at[idx[b2]], w[b2], gsem[b2])
        pd = pltpu.async_copy(
            ptab_hbm.at[pl.ds(pos_base + cc * _CHUNK, _CHUNK)], p[b3], psem[b3])
        return gd, pd

    in_desc = [None] * n_chunks
    out_desc = [None] * n_chunks
    in_desc[0] = start_loads(0)

    for cc in range(n_chunks):
        b2, b3 = cc % 2, cc % 3
        if cc + 1 < n_chunks:
            if cc - 2 >= 0:
                out_desc[cc - 2].wait()
            in_desc[cc + 1] = start_loads(cc + 1)
        gd, pd = in_desc[cc]
        gd.wait()
        pd.wait()

        def do_row(i, _, b2=b2, b3=b3):
            for j in range(jcols):
                sl = pl.ds(j * _LANES, _LANES)
                plsc.addupdate(p[b3].at[i, sl], w[b2][i, sl] * _SCALE)
            return 0

        lax.fori_loop(0, _CHUNK, do_row, 0)
        out_desc[cc] = pltpu.async_copy(
            p[b3], out_hbm.at[pl.ds(base + cc * _CHUNK, _CHUNK)], osem[b3])

    for cc in range(max(0, n_chunks - 3), n_chunks):
        out_desc[cc].wait()


def kernel(x, word_emb, pos_emb):
    batch, seq = x.shape
    nrows = batch * seq
    rows_per_w = nrows // _NW

    mesh = plsc.VectorSubcoreMesh(core_axis_name="c", subcore_axis_name="s")
    body = functools.partial(_emb_body, rows_per_w=rows_per_w)
    out = pl.kernel(
        body,
        out_type=jax.ShapeDtypeStruct((nrows, _D), jnp.float32),
        mesh=mesh,
        scratch_types=[
            pltpu.VMEM((_CHUNK,), jnp.int32),
            pltpu.VMEM((_CHUNK,), jnp.int32),
            pltpu.VMEM((_CHUNK, _D), jnp.float32),
            pltpu.VMEM((_CHUNK, _D), jnp.float32),
            pltpu.VMEM((_CHUNK, _D), jnp.float32),
            pltpu.VMEM((_CHUNK, _D), jnp.float32),
            pltpu.VMEM((_CHUNK, _D), jnp.float32),
        ] + [pltpu.SemaphoreType.DMA] * 8,
    )(x.reshape(nrows).astype(jnp.int32), word_emb, pos_emb)
    return out.reshape(batch, seq, _D)


# speedup vs baseline: 1.3965x; 1.0621x over previous
import functools
import math

import jax
import jax.numpy as jnp
from jax import lax
from jax.experimental import pallas as pl
from jax.experimental.pallas import tpu as pltpu
from jax.experimental.pallas import tpu_sc as plsc

_D = 1024
_LANES = 16
_NC = 2
_NS = 16
_NW = _NC * _NS
_SCALE = math.sqrt(_D)
_CHUNK = 16


def _emb_body(x_hbm, wtab_hbm, ptab_hbm, out_hbm,
              idx_all, w0, w1, pb0, pb1, ob0, ob1, ob2,
              g0, g1, q0, q1, o0, o1, o2,
              *, batch, seq):
    wid = lax.axis_index("s") * _NC + lax.axis_index("c")
    pos_per_w = seq // _NW
    kmax = pos_per_w // _CHUNK
    n_chunks = kmax * batch
    jcols = _D // _LANES
    xrows_per_b = seq // _CHUNK

    w = [w0, w1]
    pb = [pb0, pb1]
    ob = [ob0, ob1, ob2]
    gsem = [g0, g1]
    psem = [q0, q1]
    osem = [o0, o1, o2]

    for b in range(batch):
        pltpu.sync_copy(x_hbm.at[pl.ds(b * xrows_per_b + wid * kmax, kmax)],
                        idx_all.at[pl.ds(b * kmax, kmax)])

    def start_gather(cc):
        k, b = cc // batch, cc % batch
        return pltpu.async_copy(
            wtab_hbm.at[idx_all.at[b * kmax + k]], w[cc % 2], gsem[cc % 2])

    def start_pos(k):
        return pltpu.async_copy(
            ptab_hbm.at[pl.ds(wid * pos_per_w + k * _CHUNK, _CHUNK)],
            pb[k % 2], psem[k % 2])

    gdesc = [None] * n_chunks
    pdesc = [None] * kmax
    odesc = [None] * n_chunks
    pdesc[0] = start_pos(0)
    gdesc[0] = start_gather(0)
    if kmax > 1:
        pdesc[1] = start_pos(1)

    for cc in range(n_chunks):
        k, b = cc // batch, cc % batch
        if cc + 1 < n_chunks:
            if cc - 2 >= 0:
                odesc[cc - 2].wait()
            gdesc[cc + 1] = start_gather(cc + 1)
        if b == 0 and k >= 1 and k + 1 < kmax:
            pdesc[k + 1] = start_pos(k + 1)
        gdesc[cc].wait()
        if b == 0:
            pdesc[k].wait()

        def do_row(i, _, wb=w[cc % 2], pbk=pb[k % 2], obc=ob[cc % 3]):
            for j in range(jcols):
                sl = pl.ds(j * _LANES, _LANES)
                obc[i, sl] = wb[i, sl] * _SCALE + pbk[i, sl]
            return 0

        lax.fori_loop(0, _CHUNK, do_row, 0)
        row0 = b * seq + wid * pos_per_w + k * _CHUNK
        odesc[cc] = pltpu.async_copy(
            ob[cc % 3], out_hbm.at[pl.ds(row0, _CHUNK)], osem[cc % 3])

    for cc in range(max(0, n_chunks - 3), n_chunks):
        odesc[cc].wait()


def kernel(x, word_emb, pos_emb):
    batch, seq = x.shape
    nrows = batch * seq

    mesh = plsc.VectorSubcoreMesh(core_axis_name="c", subcore_axis_name="s")
    body = functools.partial(_emb_body, batch=batch, seq=seq)
    out = pl.kernel(
        body,
        out_type=jax.ShapeDtypeStruct((nrows, _D), jnp.float32),
        mesh=mesh,
        scratch_types=[
            pltpu.VMEM((batch * (seq // _NW // _CHUNK), _CHUNK), jnp.int32),
            pltpu.VMEM((_CHUNK, _D), jnp.float32),
            pltpu.VMEM((_CHUNK, _D), jnp.float32),
            pltpu.VMEM((_CHUNK, _D), jnp.float32),
            pltpu.VMEM((_CHUNK, _D), jnp.float32),
            pltpu.VMEM((_CHUNK, _D), jnp.float32),
            pltpu.VMEM((_CHUNK, _D), jnp.float32),
            pltpu.VMEM((_CHUNK, _D), jnp.float32),
        ] + [pltpu.SemaphoreType.DMA] * 7,
    )(x.reshape(nrows // _CHUNK, _CHUNK).astype(jnp.int32), word_emb, pos_emb)
    return out.reshape(batch, seq, _D)


# speedup vs baseline: 1.4102x; 1.0098x over previous
import functools
import math

import jax
import jax.numpy as jnp
from jax import lax
from jax.experimental import pallas as pl
from jax.experimental.pallas import tpu as pltpu
from jax.experimental.pallas import tpu_sc as plsc

_D = 1024
_LANES = 16
_NC = 2
_NS = 16
_NW = _NC * _NS
_SCALE = math.sqrt(_D)
_CHUNK = 16


def _emb_body(x_hbm, wtab_hbm, ptab_hbm, out_hbm,
              idx_all, w0, w1, w2, pb0, pb1, ob0, ob1,
              g0, g1, g2, q0, q1, o0, o1,
              *, batch, seq):
    wid = lax.axis_index("s") * _NC + lax.axis_index("c")
    pos_per_w = seq // _NW
    kmax = pos_per_w // _CHUNK
    n_chunks = kmax * batch
    jcols = _D // _LANES
    xrows_per_b = seq // _CHUNK

    w = [w0, w1, w2]
    pb = [pb0, pb1]
    ob = [ob0, ob1]
    gsem = [g0, g1, g2]
    psem = [q0, q1]
    osem = [o0, o1]

    for b in range(batch):
        pltpu.sync_copy(x_hbm.at[pl.ds(b * xrows_per_b + wid * kmax, kmax)],
                        idx_all.at[pl.ds(b * kmax, kmax)])

    def start_gather(cc):
        k, b = cc // batch, cc % batch
        return pltpu.async_copy(
            wtab_hbm.at[idx_all.at[b * kmax + k]], w[cc % 3], gsem[cc % 3])

    def start_pos(k):
        return pltpu.async_copy(
            ptab_hbm.at[pl.ds(wid * pos_per_w + k * _CHUNK, _CHUNK)],
            pb[k % 2], psem[k % 2])

    gdesc = [None] * n_chunks
    pdesc = [None] * kmax
    odesc = [None] * n_chunks
    pdesc[0] = start_pos(0)
    gdesc[0] = start_gather(0)
    if kmax > 1:
        pdesc[1] = start_pos(1)
    if n_chunks > 1:
        gdesc[1] = start_gather(1)

    for cc in range(n_chunks):
        k, b = cc // batch, cc % batch
        if cc + 2 < n_chunks:
            gdesc[cc + 2] = start_gather(cc + 2)
        if b == 0 and k >= 1 and k + 1 < kmax:
            pdesc[k + 1] = start_pos(k + 1)
        gdesc[cc].wait()
        if b == 0:
            pdesc[k].wait()
        if cc - 2 >= 0:
            odesc[cc - 2].wait()

        def do_row(i, _, wb=w[cc % 3], pbk=pb[k % 2], obc=ob[cc % 2]):
            for j in range(jcols):
                sl = pl.ds(j * _LANES, _LANES)
                obc[i, sl] = wb[i, sl] * _SCALE + pbk[i, sl]
            return 0

        lax.fori_loop(0, _CHUNK, do_row, 0)

        row0 = b * seq + wid * pos_per_w + k * _CHUNK
        odesc[cc] = pltpu.async_copy(
            ob[cc % 2], out_hbm.at[pl.ds(row0, _CHUNK)], osem[cc % 2])

    for cc in range(max(0, n_chunks - 2), n_chunks):
        odesc[cc].wait()


def kernel(x, word_emb, pos_emb):
    batch, seq = x.shape
    nrows = batch * seq

    mesh = plsc.VectorSubcoreMesh(core_axis_name="c", subcore_axis_name="s")
    body = functools.partial(_emb_body, batch=batch, seq=seq)
    out = pl.kernel(
        body,
        out_type=jax.ShapeDtypeStruct((nrows, _D), jnp.float32),
        mesh=mesh,
        scratch_types=[
            pltpu.VMEM((batch * (seq // _NW // _CHUNK), _CHUNK), jnp.int32),
            pltpu.VMEM((_CHUNK, _D), jnp.float32),
            pltpu.VMEM((_CHUNK, _D), jnp.float32),
            pltpu.VMEM((_CHUNK, _D), jnp.float32),
            pltpu.VMEM((_CHUNK, _D), jnp.float32),
            pltpu.VMEM((_CHUNK, _D), jnp.float32),
            pltpu.VMEM((_CHUNK, _D), jnp.float32),
            pltpu.VMEM((_CHUNK, _D), jnp.float32),
        ] + [pltpu.SemaphoreType.DMA] * 7,
    )(x.reshape(nrows // _CHUNK, _CHUNK).astype(jnp.int32), word_emb, pos_emb)
    return out.reshape(batch, seq, _D)


# speedup vs baseline: 1.7035x; 1.2080x over previous
import functools
import math

import jax
import jax.numpy as jnp
from jax import lax
from jax.experimental import pallas as pl
from jax.experimental.pallas import tpu as pltpu
from jax.experimental.pallas import tpu_sc as plsc

_D = 1024
_LANES = 16
_NC = 2
_NS = 16
_NW = _NC * _NS
_SCALE = math.sqrt(_D)
_CHUNK = 16


def _emb_body(x_hbm, wtab_hbm, ptab_hbm, out_hbm,
              idx_all, w0, w1, w2, pb0, pb1, ob0, ob1,
              g0, g1, g2, q0, q1, o0, o1,
              *, batch, seq):
    wid = lax.axis_index("s") * _NC + lax.axis_index("c")
    pos_per_w = seq // _NW
    kmax = pos_per_w // _CHUNK
    n_chunks = kmax * batch
    jcols = _D // _LANES
    xrows_per_b = seq // _CHUNK

    w = [w0, w1, w2]
    pb = [pb0, pb1]
    ob = [ob0, ob1]
    gsem = [g0, g1, g2]
    psem = [q0, q1]
    osem = [o0, o1]

    for b in range(batch):
        pltpu.sync_copy(x_hbm.at[pl.ds(b * xrows_per_b + wid * kmax, kmax)],
                        idx_all.at[pl.ds(b * kmax, kmax)])

    def start_gather(cc):
        k, b = cc // batch, cc % batch
        return pltpu.async_copy(
            wtab_hbm.at[idx_all.at[b * kmax + k]], w[cc % 3], gsem[cc % 3])

    def start_pos(k):
        return pltpu.async_copy(
            ptab_hbm.at[pl.ds(wid * pos_per_w + k * _CHUNK, _CHUNK)],
            pb[k % 2], psem[k % 2])

    gdesc = [None] * n_chunks
    pdesc = [None] * kmax
    odesc = [None] * n_chunks
    pdesc[0] = start_pos(0)
    gdesc[0] = start_gather(0)
    if kmax > 1:
        pdesc[1] = start_pos(1)
    if n_chunks > 1:
        gdesc[1] = start_gather(1)

    for cc in range(n_chunks):
        k, b = cc // batch, cc % batch
        if cc + 2 < n_chunks:
            gdesc[cc + 2] = start_gather(cc + 2)
        if b == 0 and k >= 1 and k + 1 < kmax:
            pdesc[k + 1] = start_pos(k + 1)
        gdesc[cc].wait()
        if b == 0:
            pdesc[k].wait()
        if cc - 2 >= 0:
            odesc[cc - 2].wait()

        def do_row(i, _, wb=w[cc % 3], pbk=pb[k % 2], obc=ob[cc % 2]):
            @plsc.parallel_loop(0, jcols, unroll=4)
            def do_j(j):
                sl = pl.ds(j * _LANES, _LANES)
                obc[i, sl] = wb[i, sl] * _SCALE + pbk[i, sl]
            return 0

        lax.fori_loop(0, _CHUNK, do_row, 0)

        row0 = b * seq + wid * pos_per_w + k * _CHUNK
        odesc[cc] = pltpu.async_copy(
            ob[cc % 2], out_hbm.at[pl.ds(row0, _CHUNK)], osem[cc % 2])

    for cc in range(max(0, n_chunks - 2), n_chunks):
        odesc[cc].wait()


def kernel(x, word_emb, pos_emb):
    batch, seq = x.shape
    nrows = batch * seq

    mesh = plsc.VectorSubcoreMesh(core_axis_name="c", subcore_axis_name="s")
    body = functools.partial(_emb_body, batch=batch, seq=seq)
    out = pl.kernel(
        body,
        out_type=jax.ShapeDtypeStruct((nrows, _D), jnp.float32),
        mesh=mesh,
        scratch_types=[
            pltpu.VMEM((batch * (seq // _NW // _CHUNK), _CHUNK), jnp.int32),
            pltpu.VMEM((_CHUNK, _D), jnp.float32),
            pltpu.VMEM((_CHUNK, _D), jnp.float32),
            pltpu.VMEM((_CHUNK, _D), jnp.float32),
            pltpu.VMEM((_CHUNK, _D), jnp.float32),
            pltpu.VMEM((_CHUNK, _D), jnp.float32),
            pltpu.VMEM((_CHUNK, _D), jnp.float32),
            pltpu.VMEM((_CHUNK, _D), jnp.float32),
        ] + [pltpu.SemaphoreType.DMA] * 7,
    )(x.reshape(nrows // _CHUNK, _CHUNK).astype(jnp.int32), word_emb, pos_emb)
    return out.reshape(batch, seq, _D)
